# Initial kernel scaffold; baseline (speedup 1.0000x reference)
import jax
import jax.numpy as jnp
from jax.experimental import pallas as pl


def kernel(x, p1, W, gamma, beta):
    raise NotImplementedError("write your pallas kernel here")


# speedup vs baseline: 8.8237x; 8.8237x over previous
import functools

import jax
import jax.numpy as jnp
from jax import lax
from jax.experimental import pallas as pl
from jax.experimental.pallas import tpu as pltpu
from jax.experimental.pallas import tpu_sc as plsc

_B, _N, _CIN, _COUT, _K = 4, 8192, 64, 128, 16
_M = _N // 4
_SUB, _LAN = 8, _N // 8
_TQ = 128
_ROWS_MLP = 512
_BIGI = 2 ** 30


def _fps_body(p_ref, p2_ref):
    px = p_ref[0, 0]
    py = p_ref[0, 1]
    pz = p_ref[0, 2]
    row = lax.broadcasted_iota(jnp.int32, (_SUB, _LAN), 0)
    col = lax.broadcasted_iota(jnp.int32, (_SUB, _LAN), 1)
    lin = row * _LAN + col

    def body(i, state):
        dists, far = state
        msk = lin == far
        cx = jnp.sum(jnp.where(msk, px, 0.0))
        cy = jnp.sum(jnp.where(msk, py, 0.0))
        cz = jnp.sum(jnp.where(msk, pz, 0.0))
        cvec = jnp.concatenate(
            [cx.reshape(1, 1), cy.reshape(1, 1), cz.reshape(1, 1)], axis=1)
        p2_ref[0, pl.ds(i, 1), :] = cvec
        dx = px - cx
        dy = py - cy
        dz = pz - cz
        d = dx * dx + dy * dy + dz * dz
        dists = jnp.minimum(dists, d)
        mx = jnp.max(dists)
        sel = jnp.where(dists == mx, lin, jnp.int32(_BIGI))
        far = jnp.min(sel)
        return dists, far

    dists0 = jnp.full((_SUB, _LAN), 1e10, jnp.float32)
    lax.fori_loop(0, _M, body, (dists0, jnp.int32(0)))


def _run_fps(p1r):
    return pl.pallas_call(
        _fps_body,
        grid=(_B,),
        in_specs=[pl.BlockSpec((1, 3, _SUB, _LAN), lambda b: (b, 0, 0, 0))],
        out_specs=pl.BlockSpec((1, _M, 3), lambda b: (b, 0, 0)),
        out_shape=jax.ShapeDtypeStruct((_B, _M, 3), jnp.float32),
    )(p1r)


def _mlp_body(x_ref, wt_ref, gb_ref, h_ref, sb_ref, acc_ref):
    i = pl.program_id(0)
    h = jnp.dot(x_ref[...], wt_ref[...], preferred_element_type=jnp.float32)
    h_ref[...] = h
    s = jnp.sum(h, axis=0, keepdims=True)
    ss = jnp.sum(h * h, axis=0, keepdims=True)

    @pl.when(i == 0)
    def _():
        acc_ref[0:1, :] = s
        acc_ref[1:2, :] = ss

    @pl.when(i > 0)
    def _():
        acc_ref[0:1, :] = acc_ref[0:1, :] + s
        acc_ref[1:2, :] = acc_ref[1:2, :] + ss

    @pl.when(i == pl.num_programs(0) - 1)
    def _():
        inv_n = 1.0 / (_B * _N)
        mean = acc_ref[0:1, :] * inv_n
        var = acc_ref[1:2, :] * inv_n - mean * mean
        scale = gb_ref[0:1, :] / jnp.sqrt(var + 1e-5)
        bias = gb_ref[1:2, :] - mean * scale
        sb_ref[0:1, :] = scale
        sb_ref[1:2, :] = bias


def _run_mlp(x2d, wt, gb):
    nsteps = (_B * _N) // _ROWS_MLP
    return pl.pallas_call(
        _mlp_body,
        grid=(nsteps,),
        in_specs=[
            pl.BlockSpec((_ROWS_MLP, _CIN), lambda i: (i, 0)),
            pl.BlockSpec((_CIN, _COUT), lambda i: (0, 0)),
            pl.BlockSpec((2, _COUT), lambda i: (0, 0)),
        ],
        out_specs=[
            pl.BlockSpec((_ROWS_MLP, _COUT), lambda i: (i, 0)),
            pl.BlockSpec((2, _COUT), lambda i: (0, 0)),
        ],
        out_shape=[
            jax.ShapeDtypeStruct((_B * _N, _COUT), jnp.float32),
            jax.ShapeDtypeStruct((2, _COUT), jnp.float32),
        ],
        scratch_shapes=[pltpu.VMEM((2, _COUT), jnp.float32)],
    )(x2d, wt, gb)


def _knn_body(p1t_ref, p2_ref, nbr_ref):
    b = pl.program_id(0)
    px = p1t_ref[0, 0:1, :]
    py = p1t_ref[0, 1:2, :]
    pz = p1t_ref[0, 2:3, :]
    rr = px * px + py * py + pz * pz
    q3 = p2_ref[0]
    qx = q3[:, 0:1]
    qy = q3[:, 1:2]
    qz = q3[:, 2:3]
    qq = qx * qx + qy * qy + qz * qz
    cross = lax.dot_general(q3, p1t_ref[0], (((1,), (0,)), ((), ())),
                            preferred_element_type=jnp.float32)
    d = qq - 2.0 * cross + rr
    li = lax.broadcasted_iota(jnp.int32, (1, _N), 1)
    offs = b * _N
    for k in range(_K):
        m = jnp.min(d, axis=1, keepdims=True)
        sel = jnp.where(d == m, li, jnp.int32(_BIGI))
        idx = jnp.min(sel, axis=1, keepdims=True)
        nbr_ref[0, :, k:k + 1] = idx + offs
        d = jnp.where(li == idx, jnp.inf, d)


def _run_knn(p1t, p2):
    return pl.pallas_call(
        _knn_body,
        grid=(_B, _M // _TQ),
        in_specs=[
            pl.BlockSpec((1, 3, _N), lambda b, m: (b, 0, 0)),
            pl.BlockSpec((1, _TQ, 3), lambda b, m: (b, m, 0)),
        ],
        out_specs=pl.BlockSpec((1, _TQ, _K), lambda b, m: (b, m, 0)),
        out_shape=jax.ShapeDtypeStruct((_B, _M, _K), jnp.int32),
    )(p1t, p2)


_NW = 32
_QPW = (_B * _M) // _NW
_CH = 8


def _pool_body(nbr_hbm, h_hbm, sb_hbm, y_hbm, idx_v, rows_v, ybuf_v, sb_v,
               sem):
    wid = lax.axis_index("s") * 2 + lax.axis_index("c")
    base_q = wid * _QPW
    pltpu.sync_copy(nbr_hbm.at[pl.ds(base_q * _K, _QPW * _K)], idx_v)
    pltpu.sync_copy(sb_hbm.at[pl.ds(0, 2 * _COUT)], sb_v)

    def chunk(c, carry):
        pltpu.async_copy(
            h_hbm.at[idx_v.at[pl.ds(c * _CH * _K, _CH * _K)]], rows_v,
            sem).wait()

        def per_q(q, carry_q):
            def per_g(g, carry_g):
                def per_r(r, acc):
                    return jnp.maximum(acc, rows_v[q * _K + r,
                                                   pl.ds(g * 16, 16)])

                acc = lax.fori_loop(1, _K, per_r,
                                    rows_v[q * _K, pl.ds(g * 16, 16)])
                sc = sb_v[pl.ds(g * 16, 16)]
                bi = sb_v[pl.ds(_COUT + g * 16, 16)]
                ybuf_v[q, pl.ds(g * 16, 16)] = jnp.maximum(
                    acc * sc + bi, 0.0)
                return carry_g

            return lax.fori_loop(0, _COUT // 16, per_g, carry_q)

        lax.fori_loop(0, _CH, per_q, 0)
        pltpu.sync_copy(ybuf_v, y_hbm.at[pl.ds(base_q + c * _CH, _CH)])
        return carry

    lax.fori_loop(0, _QPW // _CH, chunk, 0)


def _run_pool(nbr_flat, h2d, sb_flat):
    mesh = plsc.VectorSubcoreMesh(core_axis_name="c", subcore_axis_name="s")
    f = pl.kernel(
        _pool_body,
        mesh=mesh,
        out_type=jax.ShapeDtypeStruct((_B * _M, _COUT), jnp.float32),
        scratch_types=[
            pltpu.VMEM((_QPW * _K,), jnp.int32),
            pltpu.VMEM((_CH * _K, _COUT), jnp.float32),
            pltpu.VMEM((_CH, _COUT), jnp.float32),
            pltpu.VMEM((2 * _COUT,), jnp.float32),
            pltpu.SemaphoreType.DMA,
        ],
    )
    return f(nbr_flat, h2d, sb_flat)


def kernel(x, p1, W, gamma, beta):
    p1t = jnp.transpose(p1, (0, 2, 1))
    p1r = p1t.reshape(_B, 3, _SUB, _LAN)
    p2 = _run_fps(p1r)

    x2d = x.reshape(_B * _N, _CIN)
    wt = jnp.transpose(W, (1, 0))
    gb = jnp.stack([gamma, beta])
    h2d, sb = _run_mlp(x2d, wt, gb)

    nbr = _run_knn(p1t, p2)
    y2d = _run_pool(nbr.reshape(-1), h2d, sb.reshape(-1))
    return (y2d.reshape(_B, _M, _COUT), p2)


# speedup vs baseline: 13.6906x; 1.5516x over previous
import functools

import jax
import jax.numpy as jnp
from jax import lax
from jax.experimental import pallas as pl
from jax.experimental.pallas import tpu as pltpu
from jax.experimental.pallas import tpu_sc as plsc

_B, _N, _CIN, _COUT, _K = 4, 8192, 64, 128, 16
_M = _N // 4
_SUB, _LAN = 8, _N // 8
_TQ = 128
_ROWS_MLP = 512
_BIGI = 2 ** 30


def _tourn(ta, tb):
    take = (tb[0] > ta[0]) | ((tb[0] == ta[0]) & (tb[4] < ta[4]))
    return tuple(jnp.where(take, b, a) for a, b in zip(ta, tb))


def _argmax_tuple(d, px, py, pz, lin):
    t = tuple(a.reshape(_SUB, _LAN // 128, 128) for a in (d, px, py, pz, lin))
    for half in (4, 2, 1):
        t = _tourn(tuple(a[:, :half] for a in t),
                   tuple(a[:, half:2 * half] for a in t))
    t = tuple(a.reshape(_SUB, 128) for a in t)
    for k in (64, 32, 16, 8, 4, 2, 1):
        t = _tourn(t, tuple(pltpu.roll(a, k, 1) for a in t))
    for k in (4, 2, 1):
        t = _tourn(t, tuple(pltpu.roll(a, k, 0) for a in t))
    return (t[1][0:1, 0:1], t[2][0:1, 0:1], t[3][0:1, 0:1])


def _fps_body(p_ref, *p2_refs):
    px = [p_ref[b, 0] for b in range(_B)]
    py = [p_ref[b, 1] for b in range(_B)]
    pz = [p_ref[b, 2] for b in range(_B)]
    row = lax.broadcasted_iota(jnp.int32, (_SUB, _LAN), 0)
    col = lax.broadcasted_iota(jnp.int32, (_SUB, _LAN), 1)
    lin = row * _LAN + col

    def body(i, state):
        dists, cents = state
        new_d, new_c = [], []
        for b in range(_B):
            cx, cy, cz = cents[b]
            p2_refs[b][pl.ds(i, 1), :] = jnp.concatenate([cx, cy, cz],
                                                         axis=1)
            dx = px[b] - cx
            dy = py[b] - cy
            dz = pz[b] - cz
            d = dx * dx + dy * dy + dz * dz
            db = jnp.minimum(dists[b], d)
            new_d.append(db)
            new_c.append(_argmax_tuple(db, px[b], py[b], pz[b], lin))
        return tuple(new_d), tuple(new_c)

    dists0 = tuple(jnp.full((_SUB, _LAN), 1e10, jnp.float32)
                   for _ in range(_B))
    cents0 = tuple((px[b][0:1, 0:1], py[b][0:1, 0:1], pz[b][0:1, 0:1])
                   for b in range(_B))
    lax.fori_loop(0, _M, body, (dists0, cents0))


def _run_fps(p1r):
    outs = pl.pallas_call(
        _fps_body,
        in_specs=[pl.BlockSpec((_B, 3, _SUB, _LAN), lambda: (0, 0, 0, 0))],
        out_specs=[pl.BlockSpec((_M, 3), lambda: (0, 0))] * _B,
        out_shape=[jax.ShapeDtypeStruct((_M, 3), jnp.float32)] * _B,
    )(p1r)
    return jnp.stack(outs)


def _mlp_body(x_ref, wt_ref, gb_ref, h_ref, sb_ref, acc_ref):
    i = pl.program_id(0)
    h = jnp.dot(x_ref[...], wt_ref[...], preferred_element_type=jnp.float32)
    h_ref[...] = h
    s = jnp.sum(h, axis=0, keepdims=True)
    ss = jnp.sum(h * h, axis=0, keepdims=True)

    @pl.when(i == 0)
    def _():
        acc_ref[0:1, :] = s
        acc_ref[1:2, :] = ss

    @pl.when(i > 0)
    def _():
        acc_ref[0:1, :] = acc_ref[0:1, :] + s
        acc_ref[1:2, :] = acc_ref[1:2, :] + ss

    @pl.when(i == pl.num_programs(0) - 1)
    def _():
        inv_n = 1.0 / (_B * _N)
        mean = acc_ref[0:1, :] * inv_n
        var = acc_ref[1:2, :] * inv_n - mean * mean
        scale = gb_ref[0:1, :] / jnp.sqrt(var + 1e-5)
        bias = gb_ref[1:2, :] - mean * scale
        sb_ref[0:1, :] = scale
        sb_ref[1:2, :] = bias


def _run_mlp(x2d, wt, gb):
    nsteps = (_B * _N) // _ROWS_MLP
    return pl.pallas_call(
        _mlp_body,
        grid=(nsteps,),
        in_specs=[
            pl.BlockSpec((_ROWS_MLP, _CIN), lambda i: (i, 0)),
            pl.BlockSpec((_CIN, _COUT), lambda i: (0, 0)),
            pl.BlockSpec((2, _COUT), lambda i: (0, 0)),
        ],
        out_specs=[
            pl.BlockSpec((_ROWS_MLP, _COUT), lambda i: (i, 0)),
            pl.BlockSpec((2, _COUT), lambda i: (0, 0)),
        ],
        out_shape=[
            jax.ShapeDtypeStruct((_B * _N, _COUT), jnp.float32),
            jax.ShapeDtypeStruct((2, _COUT), jnp.float32),
        ],
        scratch_shapes=[pltpu.VMEM((2, _COUT), jnp.float32)],
    )(x2d, wt, gb)


def _knn_body(p1t_ref, p2_ref, nbr_ref):
    b = pl.program_id(0)
    px = p1t_ref[0, 0:1, :]
    py = p1t_ref[0, 1:2, :]
    pz = p1t_ref[0, 2:3, :]
    rr = px * px + py * py + pz * pz
    q3 = p2_ref[0]
    qx = q3[:, 0:1]
    qy = q3[:, 1:2]
    qz = q3[:, 2:3]
    qq = qx * qx + qy * qy + qz * qz
    cross = lax.dot_general(q3, p1t_ref[0], (((1,), (0,)), ((), ())),
                            preferred_element_type=jnp.float32)
    d = qq - 2.0 * cross + rr
    li = lax.broadcasted_iota(jnp.int32, (1, _N), 1)
    offs = b * _N
    for k in range(_K):
        m = jnp.min(d, axis=1, keepdims=True)
        sel = jnp.where(d == m, li, jnp.int32(_BIGI))
        idx = jnp.min(sel, axis=1, keepdims=True)
        nbr_ref[0, :, k:k + 1] = idx + offs
        d = jnp.where(li == idx, jnp.inf, d)


def _run_knn(p1t, p2):
    return pl.pallas_call(
        _knn_body,
        grid=(_B, _M // _TQ),
        in_specs=[
            pl.BlockSpec((1, 3, _N), lambda b, m: (b, 0, 0)),
            pl.BlockSpec((1, _TQ, 3), lambda b, m: (b, m, 0)),
        ],
        out_specs=pl.BlockSpec((1, _TQ, _K), lambda b, m: (b, m, 0)),
        out_shape=jax.ShapeDtypeStruct((_B, _M, _K), jnp.int32),
    )(p1t, p2)


_NW = 32
_QPW = (_B * _M) // _NW
_CH = 8


def _pool_body(nbr_hbm, h_hbm, sb_hbm, y_hbm, idx_v, rows_v, ybuf_v, sb_v,
               sem):
    wid = lax.axis_index("s") * 2 + lax.axis_index("c")
    base_q = wid * _QPW
    pltpu.sync_copy(nbr_hbm.at[pl.ds(base_q * _K, _QPW * _K)], idx_v)
    pltpu.sync_copy(sb_hbm.at[pl.ds(0, 2 * _COUT)], sb_v)

    def chunk(c, carry):
        pltpu.async_copy(
            h_hbm.at[idx_v.at[pl.ds(c * _CH * _K, _CH * _K)]], rows_v,
            sem).wait()

        def per_q(q, carry_q):
            def per_g(g, carry_g):
                def per_r(r, acc):
                    return jnp.maximum(acc, rows_v[q * _K + r,
                                                   pl.ds(g * 16, 16)])

                acc = lax.fori_loop(1, _K, per_r,
                                    rows_v[q * _K, pl.ds(g * 16, 16)])
                sc = sb_v[pl.ds(g * 16, 16)]
                bi = sb_v[pl.ds(_COUT + g * 16, 16)]
                ybuf_v[q, pl.ds(g * 16, 16)] = jnp.maximum(
                    acc * sc + bi, 0.0)
                return carry_g

            return lax.fori_loop(0, _COUT // 16, per_g, carry_q)

        lax.fori_loop(0, _CH, per_q, 0)
        pltpu.sync_copy(ybuf_v, y_hbm.at[pl.ds(base_q + c * _CH, _CH)])
        return carry

    lax.fori_loop(0, _QPW // _CH, chunk, 0)


def _run_pool(nbr_flat, h2d, sb_flat):
    mesh = plsc.VectorSubcoreMesh(core_axis_name="c", subcore_axis_name="s")
    f = pl.kernel(
        _pool_body,
        mesh=mesh,
        out_type=jax.ShapeDtypeStruct((_B * _M, _COUT), jnp.float32),
        scratch_types=[
            pltpu.VMEM((_QPW * _K,), jnp.int32),
            pltpu.VMEM((_CH * _K, _COUT), jnp.float32),
            pltpu.VMEM((_CH, _COUT), jnp.float32),
            pltpu.VMEM((2 * _COUT,), jnp.float32),
            pltpu.SemaphoreType.DMA,
        ],
    )
    return f(nbr_flat, h2d, sb_flat)


def kernel(x, p1, W, gamma, beta):
    p1t = jnp.transpose(p1, (0, 2, 1))
    p1r = p1t.reshape(_B, 3, _SUB, _LAN)
    p2 = _run_fps(p1r)

    x2d = x.reshape(_B * _N, _CIN)
    wt = jnp.transpose(W, (1, 0))
    gb = jnp.stack([gamma, beta])
    h2d, sb = _run_mlp(x2d, wt, gb)

    nbr = _run_knn(p1t, p2)
    y2d = _run_pool(nbr.reshape(-1), h2d, sb.reshape(-1))
    return (y2d.reshape(_B, _M, _COUT), p2)
